# baseline (device time: 53326 ns/iter reference)
import os

import jax
import jax.numpy as jnp
from jax import lax
from jax.experimental import pallas as pl
from jax.experimental.pallas import tpu as pltpu

N_DEV = 4

_VARIANT = os.environ.get("KVAR", "full")
_DO_COMM = _VARIANT != "nocomm"
_DO_DOT = _VARIANT != "nodot"
_DO_W = _VARIANT != "now"


def kernel(x, w_mat, scale_x, scale_w):
    M, k_per = x.shape
    K, N = w_mat.shape
    m_per = M // N_DEV

    x8 = x.astype(jnp.float8_e5m2).reshape(N_DEV, m_per, k_per)
    w32 = w_mat.reshape(N_DEV, k_per, N)
    s = (scale_x * scale_w).reshape(1, 1)

    def body(x8_ref, w_hbm, s_ref, out_ref,
             comm_ref, wbuf, w8buf,
             send_sems, recv_sems, w_sems):
        my = lax.axis_index("i")

        sends = []
        if _DO_COMM:
            barrier = pltpu.get_barrier_semaphore()
            for d in range(1, N_DEV):
                pl.semaphore_signal(
                    barrier, inc=1,
                    device_id=((my + d) % N_DEV,),
                    device_id_type=pltpu.DeviceIdType.MESH,
                )
            pl.semaphore_wait(barrier, N_DEV - 1)

            for d in range(1, N_DEV):
                tgt = (my + d) % N_DEV
                rdma = pltpu.make_async_remote_copy(
                    src_ref=x8_ref.at[tgt],
                    dst_ref=comm_ref.at[d - 1],
                    send_sem=send_sems.at[d - 1],
                    recv_sem=recv_sems.at[d - 1],
                    device_id=(tgt,),
                    device_id_type=pltpu.DeviceIdType.MESH,
                )
                rdma.start()
                sends.append(rdma)

        w_order = [my] + [(my - d) % N_DEV for d in (1, 3, 2)]
        w_dmas = []
        if _DO_W:
            for k, blk in enumerate(w_order[:2]):
                dma = pltpu.make_async_copy(w_hbm.at[blk], wbuf.at[k % 2],
                                            w_sems.at[k % 2])
                dma.start()
                w_dmas.append(dma)

        def consume_w(k):
            slot = k % 2
            if not _DO_W:
                return slot
            w_dmas[k].wait()
            w8buf[slot] = wbuf[slot].astype(jnp.float8_e5m2)
            if k + 2 < N_DEV:
                dma = pltpu.make_async_copy(w_hbm.at[w_order[k + 2]],
                                            wbuf.at[slot], w_sems.at[slot])
                dma.start()
                w_dmas.append(dma)
            return slot

        slot = consume_w(0)
        if _DO_DOT:
            out_ref[...] = jnp.dot(
                x8_ref[my], w8buf[slot], preferred_element_type=jnp.float32
            )
        else:
            out_ref[...] = w8buf[slot].astype(jnp.float32)
            out_ref[:, :k_per] += x8_ref[my].astype(jnp.float32)

        for k, d in enumerate((1, 3, 2), start=1):
            if _DO_COMM:
                recv = pltpu.make_async_remote_copy(
                    src_ref=comm_ref.at[d - 1],
                    dst_ref=comm_ref.at[d - 1],
                    send_sem=send_sems.at[d - 1],
                    recv_sem=recv_sems.at[d - 1],
                    device_id=(my,),
                    device_id_type=pltpu.DeviceIdType.MESH,
                )
                recv.wait_recv()
            slot = consume_w(k)
            if _DO_DOT:
                acc = out_ref[...] + jnp.dot(
                    comm_ref[d - 1], w8buf[slot],
                    preferred_element_type=jnp.float32,
                )
                out_ref[...] = acc * s_ref[0, 0] if k == N_DEV - 1 else acc
            else:
                out_ref[...] += w8buf[slot].astype(jnp.float32)
                out_ref[:, :k_per] += comm_ref[d - 1].astype(jnp.float32)

        for rdma in sends:
            rdma.wait_send()

    return pl.pallas_call(
        body,
        out_shape=jax.ShapeDtypeStruct((m_per, N), jnp.float32),
        in_specs=[
            pl.BlockSpec(memory_space=pltpu.VMEM),
            pl.BlockSpec(memory_space=pltpu.MemorySpace.HBM),
            pl.BlockSpec(memory_space=pltpu.SMEM),
        ],
        out_specs=pl.BlockSpec(memory_space=pltpu.VMEM),
        scratch_shapes=[
            pltpu.VMEM((N_DEV - 1, m_per, k_per), jnp.float8_e5m2),
            pltpu.VMEM((2, k_per, N), jnp.float32),
            pltpu.VMEM((2, k_per, N), jnp.float8_e5m2),
            pltpu.SemaphoreType.DMA((N_DEV - 1,)),
            pltpu.SemaphoreType.DMA((N_DEV - 1,)),
            pltpu.SemaphoreType.DMA((2,)),
        ],
        compiler_params=pltpu.CompilerParams(
            collective_id=0 if _DO_COMM else None,
            vmem_limit_bytes=100 * 1024 * 1024,
        ),
    )(x8, w32, s)


# device time: 53276 ns/iter; 1.0009x vs baseline; 1.0009x over previous
import os

import jax
import jax.numpy as jnp
from jax import lax
from jax.experimental import pallas as pl
from jax.experimental.pallas import tpu as pltpu

N_DEV = 4

_VARIANT = os.environ.get("KVAR", "full")
_DO_COMM = _VARIANT != "nocomm"
_DO_DOT = _VARIANT != "nodot"
_DO_W = _VARIANT != "now"


def kernel(x, w_mat, scale_x, scale_w):
    M, k_per = x.shape
    K, N = w_mat.shape
    m_per = M // N_DEV

    x8 = x.astype(jnp.float8_e5m2)
    s = (scale_x * scale_w).reshape(1, 1)

    def body(x8_ref, w_hbm, s_ref, out_ref,
             comm_ref, wbuf, w8buf,
             send_sems, recv_sems, w_sems):
        my = lax.axis_index("i")

        sends = []
        if _DO_COMM:
            barrier = pltpu.get_barrier_semaphore()
            for d in range(1, N_DEV):
                pl.semaphore_signal(
                    barrier, inc=1,
                    device_id=((my + d) % N_DEV,),
                    device_id_type=pltpu.DeviceIdType.MESH,
                )
            pl.semaphore_wait(barrier, N_DEV - 1)

            for d in range(1, N_DEV):
                tgt = (my + d) % N_DEV
                rdma = pltpu.make_async_remote_copy(
                    src_ref=x8_ref.at[pl.ds(tgt * m_per, m_per), :],
                    dst_ref=comm_ref.at[d - 1],
                    send_sem=send_sems.at[d - 1],
                    recv_sem=recv_sems.at[d - 1],
                    device_id=(tgt,),
                    device_id_type=pltpu.DeviceIdType.MESH,
                )
                rdma.start()
                sends.append(rdma)

        w_order = [my] + [(my - d) % N_DEV for d in (1, 3, 2)]
        w_dmas = []
        def w_src(blk):
            return w_hbm.at[pl.ds(blk * k_per, k_per), :]

        if _DO_W:
            for k, blk in enumerate(w_order[:2]):
                dma = pltpu.make_async_copy(w_src(blk), wbuf.at[k % 2],
                                            w_sems.at[k % 2])
                dma.start()
                w_dmas.append(dma)

        def consume_w(k):
            slot = k % 2
            if not _DO_W:
                return slot
            w_dmas[k].wait()
            w8buf[slot] = wbuf[slot].astype(jnp.float8_e5m2)
            if k + 2 < N_DEV:
                dma = pltpu.make_async_copy(w_src(w_order[k + 2]),
                                            wbuf.at[slot], w_sems.at[slot])
                dma.start()
                w_dmas.append(dma)
            return slot

        slot = consume_w(0)
        x_own = x8_ref[pl.ds(my * m_per, m_per), :]
        if _DO_DOT:
            out_ref[...] = jnp.dot(
                x_own, w8buf[slot], preferred_element_type=jnp.float32
            )
        else:
            out_ref[...] = w8buf[slot].astype(jnp.float32)
            out_ref[:, :k_per] += x_own.astype(jnp.float32)

        for k, d in enumerate((1, 3, 2), start=1):
            if _DO_COMM:
                recv = pltpu.make_async_remote_copy(
                    src_ref=comm_ref.at[d - 1],
                    dst_ref=comm_ref.at[d - 1],
                    send_sem=send_sems.at[d - 1],
                    recv_sem=recv_sems.at[d - 1],
                    device_id=(my,),
                    device_id_type=pltpu.DeviceIdType.MESH,
                )
                recv.wait_recv()
            slot = consume_w(k)
            if _DO_DOT:
                acc = out_ref[...] + jnp.dot(
                    comm_ref[d - 1], w8buf[slot],
                    preferred_element_type=jnp.float32,
                )
                out_ref[...] = acc * s_ref[0, 0] if k == N_DEV - 1 else acc
            else:
                out_ref[...] += w8buf[slot].astype(jnp.float32)
                out_ref[:, :k_per] += comm_ref[d - 1].astype(jnp.float32)

        for rdma in sends:
            rdma.wait_send()

    return pl.pallas_call(
        body,
        out_shape=jax.ShapeDtypeStruct((m_per, N), jnp.float32),
        in_specs=[
            pl.BlockSpec(memory_space=pltpu.VMEM),
            pl.BlockSpec(memory_space=pltpu.MemorySpace.HBM),
            pl.BlockSpec(memory_space=pltpu.SMEM),
        ],
        out_specs=pl.BlockSpec(memory_space=pltpu.VMEM),
        scratch_shapes=[
            pltpu.VMEM((N_DEV - 1, m_per, k_per), jnp.float8_e5m2),
            pltpu.VMEM((2, k_per, N), jnp.float32),
            pltpu.VMEM((2, k_per, N), jnp.float8_e5m2),
            pltpu.SemaphoreType.DMA((N_DEV - 1,)),
            pltpu.SemaphoreType.DMA((N_DEV - 1,)),
            pltpu.SemaphoreType.DMA((2,)),
        ],
        compiler_params=pltpu.CompilerParams(
            collective_id=0 if _DO_COMM else None,
            vmem_limit_bytes=100 * 1024 * 1024,
        ),
    )(x8, w_mat, s)


# device time: 51738 ns/iter; 1.0307x vs baseline; 1.0297x over previous
import jax
import jax.numpy as jnp
from jax import lax
from jax.experimental import pallas as pl
from jax.experimental.pallas import tpu as pltpu

N_DEV = 4


def kernel(x, w_mat, scale_x, scale_w):
    M, k_per = x.shape
    K, N = w_mat.shape
    m_per = M // N_DEV

    x8 = x.astype(jnp.float8_e5m2)
    s = (scale_x * scale_w).reshape(1, 1)

    def body(x8_hbm, w_hbm, s_ref, out_hbm,
             xblk, comm_ref, wbuf, w8buf, acc_ref,
             send_sems, recv_sems, w_sems, x_sem, o_sem):
        my = lax.axis_index("i")

        w_order = [my] + [(my - d) % N_DEV for d in (1, 3, 2)]

        def w_src(blk):
            return w_hbm.at[pl.ds(blk * k_per, k_per), :]

        w_dmas = []
        for k in range(2):
            dma = pltpu.make_async_copy(w_src(w_order[k]), wbuf.at[k],
                                        w_sems.at[k])
            dma.start()
            w_dmas.append(dma)
        xdma = pltpu.make_async_copy(
            x8_hbm.at[pl.ds(my * m_per, m_per), :], xblk, x_sem)
        xdma.start()

        barrier = pltpu.get_barrier_semaphore()
        for d in range(1, N_DEV):
            pl.semaphore_signal(
                barrier, inc=1,
                device_id=((my + d) % N_DEV,),
                device_id_type=pltpu.DeviceIdType.MESH,
            )
        pl.semaphore_wait(barrier, N_DEV - 1)

        sends = []
        for d in range(1, N_DEV):
            tgt = (my + d) % N_DEV
            rdma = pltpu.make_async_remote_copy(
                src_ref=x8_hbm.at[pl.ds(tgt * m_per, m_per), :],
                dst_ref=comm_ref.at[d - 1],
                send_sem=send_sems.at[d - 1],
                recv_sem=recv_sems.at[d - 1],
                device_id=(tgt,),
                device_id_type=pltpu.DeviceIdType.MESH,
            )
            rdma.start()
            sends.append(rdma)

        def consume_w(k):
            slot = k % 2
            w_dmas[k].wait()
            w8buf[slot] = wbuf[slot].astype(jnp.float8_e5m2)
            if k + 2 < N_DEV:
                dma = pltpu.make_async_copy(w_src(w_order[k + 2]),
                                            wbuf.at[slot], w_sems.at[slot])
                dma.start()
                w_dmas.append(dma)
            return slot

        slot = consume_w(0)
        xdma.wait()
        acc_ref[...] = jnp.dot(
            xblk[...], w8buf[slot], preferred_element_type=jnp.float32
        )

        for k, d in enumerate((1, 3, 2), start=1):
            recv = pltpu.make_async_remote_copy(
                src_ref=comm_ref.at[d - 1],
                dst_ref=comm_ref.at[d - 1],
                send_sem=send_sems.at[d - 1],
                recv_sem=recv_sems.at[d - 1],
                device_id=(my,),
                device_id_type=pltpu.DeviceIdType.MESH,
            )
            recv.wait_recv()
            slot = consume_w(k)
            a = acc_ref[...] + jnp.dot(
                comm_ref[d - 1], w8buf[slot],
                preferred_element_type=jnp.float32,
            )
            acc_ref[...] = a * s_ref[0, 0] if k == N_DEV - 1 else a

        odma = pltpu.make_async_copy(acc_ref, out_hbm, o_sem)
        odma.start()
        odma.wait()
        for rdma in sends:
            rdma.wait_send()

    return pl.pallas_call(
        body,
        out_shape=jax.ShapeDtypeStruct((m_per, N), jnp.float32),
        in_specs=[
            pl.BlockSpec(memory_space=pltpu.MemorySpace.HBM),
            pl.BlockSpec(memory_space=pltpu.MemorySpace.HBM),
            pl.BlockSpec(memory_space=pltpu.SMEM),
        ],
        out_specs=pl.BlockSpec(memory_space=pltpu.MemorySpace.HBM),
        scratch_shapes=[
            pltpu.VMEM((m_per, k_per), jnp.float8_e5m2),
            pltpu.VMEM((N_DEV - 1, m_per, k_per), jnp.float8_e5m2),
            pltpu.VMEM((2, k_per, N), jnp.float32),
            pltpu.VMEM((2, k_per, N), jnp.float8_e5m2),
            pltpu.VMEM((m_per, N), jnp.float32),
            pltpu.SemaphoreType.DMA((N_DEV - 1,)),
            pltpu.SemaphoreType.DMA((N_DEV - 1,)),
            pltpu.SemaphoreType.DMA((2,)),
            pltpu.SemaphoreType.DMA,
            pltpu.SemaphoreType.DMA,
        ],
        compiler_params=pltpu.CompilerParams(
            collective_id=0,
            vmem_limit_bytes=100 * 1024 * 1024,
        ),
    )(x8, w_mat, s)


# device time: 44745 ns/iter; 1.1918x vs baseline; 1.1563x over previous
import jax
import jax.numpy as jnp
from jax import lax
from jax.experimental import pallas as pl
from jax.experimental.pallas import tpu as pltpu

N_DEV = 4
CPB = 2


def kernel(x, w_mat, scale_x, scale_w):
    M, k_per = x.shape
    K, N = w_mat.shape
    m_per = M // N_DEV
    c_rows = m_per // CPB
    n_chunks = N_DEV * CPB

    s = (scale_x * scale_w).reshape(1, 1)

    d_order = (1, 3, 2)

    def body(x_hbm, w_hbm, s_ref, out_hbm,
             xf32, x8_ref, comm_ref, wbuf, w8buf, acc_ref,
             send_sems, recv_sems, w_sems, x_sems, o_sems):
        my = lax.axis_index("i")

        w_order = [my] + [(my - d) % N_DEV for d in d_order]

        def w_src(blk):
            return w_hbm.at[pl.ds(blk * k_per, k_per), :]

        w_dmas = []
        for k in range(2):
            dma = pltpu.make_async_copy(w_src(w_order[k]), wbuf.at[k],
                                        w_sems.at[k])
            dma.start()
            w_dmas.append(dma)

        def consume_w(k):
            slot = k % 2
            w_dmas[k].wait()
            w8buf[slot] = wbuf[slot].astype(jnp.float8_e5m2)
            if k + 2 < N_DEV:
                dma = pltpu.make_async_copy(w_src(w_order[k + 2]),
                                            wbuf.at[slot], w_sems.at[slot])
                dma.start()
                w_dmas.append(dma)
            return slot

        chunks = []
        for d in d_order:
            for j in range(CPB):
                chunks.append((d, j))
        for j in range(CPB):
            chunks.append((0, j))

        def x_row0(d, j):
            tgt = (my + d) % N_DEV
            return tgt * m_per + j * c_rows

        x_dmas = []
        for i in range(2):
            d, j = chunks[i]
            dma = pltpu.make_async_copy(
                x_hbm.at[pl.ds(x_row0(d, j), c_rows), :],
                xf32.at[i % 2], x_sems.at[i % 2])
            dma.start()
            x_dmas.append(dma)

        barrier = pltpu.get_barrier_semaphore()
        for d in range(1, N_DEV):
            pl.semaphore_signal(
                barrier, inc=1,
                device_id=((my + d) % N_DEV,),
                device_id_type=pltpu.DeviceIdType.MESH,
            )
        pl.semaphore_wait(barrier, N_DEV - 1)

        sends = []
        for i, (d, j) in enumerate(chunks):
            x_dmas[i].wait()
            row0 = x_row0(d, j)
            x8_ref[pl.ds(row0, c_rows), :] = (
                xf32[i % 2].astype(jnp.float8_e5m2))
            if i + 2 < n_chunks:
                nd, nj = chunks[i + 2]
                dma = pltpu.make_async_copy(
                    x_hbm.at[pl.ds(x_row0(nd, nj), c_rows), :],
                    xf32.at[i % 2], x_sems.at[i % 2])
                dma.start()
                x_dmas.append(dma)
            if d != 0:
                idx = (d - 1) * CPB + j
                rdma = pltpu.make_async_remote_copy(
                    src_ref=x8_ref.at[pl.ds(row0, c_rows), :],
                    dst_ref=comm_ref.at[d - 1, pl.ds(j * c_rows, c_rows), :],
                    send_sem=send_sems.at[idx],
                    recv_sem=recv_sems.at[idx],
                    device_id=((my + d) % N_DEV,),
                    device_id_type=pltpu.DeviceIdType.MESH,
                )
                rdma.start()
                sends.append(rdma)

        slot = consume_w(0)
        for j in range(CPB):
            rows = pl.ds(j * c_rows, c_rows)
            acc_ref[rows, :] = jnp.dot(
                x8_ref[pl.ds(my * m_per + j * c_rows, c_rows), :],
                w8buf[slot], preferred_element_type=jnp.float32,
            )

        o_dmas = []
        for k, d in enumerate(d_order, start=1):
            slot = consume_w(k)
            for j in range(CPB):
                idx = (d - 1) * CPB + j
                rows = pl.ds(j * c_rows, c_rows)
                recv = pltpu.make_async_remote_copy(
                    src_ref=comm_ref.at[d - 1, rows, :],
                    dst_ref=comm_ref.at[d - 1, rows, :],
                    send_sem=send_sems.at[idx],
                    recv_sem=recv_sems.at[idx],
                    device_id=(my,),
                    device_id_type=pltpu.DeviceIdType.MESH,
                )
                recv.wait_recv()
                a = acc_ref[rows, :] + jnp.dot(
                    comm_ref[d - 1, rows, :], w8buf[slot],
                    preferred_element_type=jnp.float32,
                )
                if k == N_DEV - 1:
                    acc_ref[rows, :] = a * s_ref[0, 0]
                    odma = pltpu.make_async_copy(
                        acc_ref.at[rows, :], out_hbm.at[rows, :],
                        o_sems.at[j])
                    odma.start()
                    o_dmas.append(odma)
                else:
                    acc_ref[rows, :] = a

        for dma in o_dmas:
            dma.wait()
        for rdma in sends:
            rdma.wait_send()

    return pl.pallas_call(
        body,
        out_shape=jax.ShapeDtypeStruct((m_per, N), jnp.float32),
        in_specs=[
            pl.BlockSpec(memory_space=pltpu.MemorySpace.HBM),
            pl.BlockSpec(memory_space=pltpu.MemorySpace.HBM),
            pl.BlockSpec(memory_space=pltpu.SMEM),
        ],
        out_specs=pl.BlockSpec(memory_space=pltpu.MemorySpace.HBM),
        scratch_shapes=[
            pltpu.VMEM((2, c_rows, k_per), jnp.float32),
            pltpu.VMEM((M, k_per), jnp.float8_e5m2),
            pltpu.VMEM((N_DEV - 1, m_per, k_per), jnp.float8_e5m2),
            pltpu.VMEM((2, k_per, N), jnp.float32),
            pltpu.VMEM((2, k_per, N), jnp.float8_e5m2),
            pltpu.VMEM((m_per, N), jnp.float32),
            pltpu.SemaphoreType.DMA(((N_DEV - 1) * CPB,)),
            pltpu.SemaphoreType.DMA(((N_DEV - 1) * CPB,)),
            pltpu.SemaphoreType.DMA((2,)),
            pltpu.SemaphoreType.DMA((2,)),
            pltpu.SemaphoreType.DMA((CPB,)),
        ],
        compiler_params=pltpu.CompilerParams(
            collective_id=0,
            vmem_limit_bytes=100 * 1024 * 1024,
        ),
    )(x, w_mat, s)


# device time: 44631 ns/iter; 1.1948x vs baseline; 1.0026x over previous
import jax
import jax.numpy as jnp
from jax import lax
from jax.experimental import pallas as pl
from jax.experimental.pallas import tpu as pltpu

N_DEV = 4
CPB = 4


def kernel(x, w_mat, scale_x, scale_w):
    M, k_per = x.shape
    K, N = w_mat.shape
    m_per = M // N_DEV
    c_rows = m_per // CPB
    n_chunks = N_DEV * CPB

    s = (scale_x * scale_w).reshape(1, 1)

    d_order = (1, 3, 2)

    def body(x_hbm, w_hbm, s_ref, out_hbm,
             xf32, x8_ref, comm_ref, wbuf, w8buf, acc_ref,
             send_sems, recv_sems, w_sems, x_sems, o_sems):
        my = lax.axis_index("i")

        w_order = [my] + [(my - d) % N_DEV for d in d_order]

        def w_src(blk):
            return w_hbm.at[pl.ds(blk * k_per, k_per), :]

        w_dmas = []
        for k in range(2):
            dma = pltpu.make_async_copy(w_src(w_order[k]), wbuf.at[k],
                                        w_sems.at[k])
            dma.start()
            w_dmas.append(dma)

        def consume_w(k):
            slot = k % 2
            w_dmas[k].wait()
            w8buf[slot] = wbuf[slot].astype(jnp.float8_e5m2)
            if k + 2 < N_DEV:
                dma = pltpu.make_async_copy(w_src(w_order[k + 2]),
                                            wbuf.at[slot], w_sems.at[slot])
                dma.start()
                w_dmas.append(dma)
            return slot

        chunks = []
        for d in d_order:
            for j in range(CPB):
                chunks.append((d, j))
        for j in range(CPB):
            chunks.append((0, j))

        def x_row0(d, j):
            tgt = (my + d) % N_DEV
            return tgt * m_per + j * c_rows

        x_dmas = []
        for i in range(2):
            d, j = chunks[i]
            dma = pltpu.make_async_copy(
                x_hbm.at[pl.ds(x_row0(d, j), c_rows), :],
                xf32.at[i % 2], x_sems.at[i % 2])
            dma.start()
            x_dmas.append(dma)

        barrier = pltpu.get_barrier_semaphore()
        for d in range(1, N_DEV):
            pl.semaphore_signal(
                barrier, inc=1,
                device_id=((my + d) % N_DEV,),
                device_id_type=pltpu.DeviceIdType.MESH,
            )
        pl.semaphore_wait(barrier, N_DEV - 1)

        sends = []
        for i, (d, j) in enumerate(chunks):
            x_dmas[i].wait()
            row0 = x_row0(d, j)
            x8_ref[pl.ds(row0, c_rows), :] = (
                xf32[i % 2].astype(jnp.float8_e5m2))
            if i + 2 < n_chunks:
                nd, nj = chunks[i + 2]
                dma = pltpu.make_async_copy(
                    x_hbm.at[pl.ds(x_row0(nd, nj), c_rows), :],
                    xf32.at[i % 2], x_sems.at[i % 2])
                dma.start()
                x_dmas.append(dma)
            if d != 0:
                idx = (d - 1) * CPB + j
                rdma = pltpu.make_async_remote_copy(
                    src_ref=x8_ref.at[pl.ds(row0, c_rows), :],
                    dst_ref=comm_ref.at[d - 1, pl.ds(j * c_rows, c_rows), :],
                    send_sem=send_sems.at[idx],
                    recv_sem=recv_sems.at[idx],
                    device_id=((my + d) % N_DEV,),
                    device_id_type=pltpu.DeviceIdType.MESH,
                )
                rdma.start()
                sends.append(rdma)

        slot = consume_w(0)
        for j in range(CPB):
            rows = pl.ds(j * c_rows, c_rows)
            acc_ref[rows, :] = jnp.dot(
                x8_ref[pl.ds(my * m_per + j * c_rows, c_rows), :],
                w8buf[slot], preferred_element_type=jnp.float32,
            )

        o_dmas = []
        for k, d in enumerate(d_order, start=1):
            slot = consume_w(k)
            for j in range(CPB):
                idx = (d - 1) * CPB + j
                rows = pl.ds(j * c_rows, c_rows)
                recv = pltpu.make_async_remote_copy(
                    src_ref=comm_ref.at[d - 1, rows, :],
                    dst_ref=comm_ref.at[d - 1, rows, :],
                    send_sem=send_sems.at[idx],
                    recv_sem=recv_sems.at[idx],
                    device_id=(my,),
                    device_id_type=pltpu.DeviceIdType.MESH,
                )
                recv.wait_recv()
                a = acc_ref[rows, :] + jnp.dot(
                    comm_ref[d - 1, rows, :], w8buf[slot],
                    preferred_element_type=jnp.float32,
                )
                if k == N_DEV - 1:
                    acc_ref[rows, :] = a * s_ref[0, 0]
                    odma = pltpu.make_async_copy(
                        acc_ref.at[rows, :], out_hbm.at[rows, :],
                        o_sems.at[j])
                    odma.start()
                    o_dmas.append(odma)
                else:
                    acc_ref[rows, :] = a

        for dma in o_dmas:
            dma.wait()
        for rdma in sends:
            rdma.wait_send()

    return pl.pallas_call(
        body,
        out_shape=jax.ShapeDtypeStruct((m_per, N), jnp.float32),
        in_specs=[
            pl.BlockSpec(memory_space=pltpu.MemorySpace.HBM),
            pl.BlockSpec(memory_space=pltpu.MemorySpace.HBM),
            pl.BlockSpec(memory_space=pltpu.SMEM),
        ],
        out_specs=pl.BlockSpec(memory_space=pltpu.MemorySpace.HBM),
        scratch_shapes=[
            pltpu.VMEM((2, c_rows, k_per), jnp.float32),
            pltpu.VMEM((M, k_per), jnp.float8_e5m2),
            pltpu.VMEM((N_DEV - 1, m_per, k_per), jnp.float8_e5m2),
            pltpu.VMEM((2, k_per, N), jnp.float32),
            pltpu.VMEM((2, k_per, N), jnp.float8_e5m2),
            pltpu.VMEM((m_per, N), jnp.float32),
            pltpu.SemaphoreType.DMA(((N_DEV - 1) * CPB,)),
            pltpu.SemaphoreType.DMA(((N_DEV - 1) * CPB,)),
            pltpu.SemaphoreType.DMA((2,)),
            pltpu.SemaphoreType.DMA((2,)),
            pltpu.SemaphoreType.DMA((CPB,)),
        ],
        compiler_params=pltpu.CompilerParams(
            collective_id=0,
            vmem_limit_bytes=100 * 1024 * 1024,
        ),
    )(x, w_mat, s)


# device time: 31895 ns/iter; 1.6719x vs baseline; 1.3993x over previous
import os

import jax
import jax.numpy as jnp
from jax import lax
from jax.experimental import pallas as pl
from jax.experimental.pallas import tpu as pltpu

N_DEV = 4
CPB = 4
_NODIAG = os.environ.get("KVAR") == "nodiag"


def kernel(x, w_mat, scale_x, scale_w):
    M, k_per = x.shape
    K, N = w_mat.shape
    m_per = M // N_DEV
    c_rows = m_per // CPB
    n_chunks = N_DEV * CPB

    s = (scale_x * scale_w).reshape(1, 1)

    d_order = (1, 3) if _NODIAG else (1, 3, 2)

    def body(x_hbm, w_hbm, s_ref, out_hbm,
             xf32, x8_ref, comm_ref, wbuf, w8buf, acc_ref,
             send_sems, recv_sems, w_sems, x_sems, o_sems):
        my = lax.axis_index("i")

        w_order = [my] + [(my - d) % N_DEV for d in d_order]

        def w_src(blk):
            return w_hbm.at[pl.ds(blk * k_per, k_per), :]

        w_dmas = []
        for k in range(2):
            dma = pltpu.make_async_copy(w_src(w_order[k]), wbuf.at[k],
                                        w_sems.at[k])
            dma.start()
            w_dmas.append(dma)

        def consume_w(k):
            slot = k % 2
            w_dmas[k].wait()
            w8buf[slot] = wbuf[slot].astype(jnp.float8_e5m2)
            if k + 2 < len(w_order):
                dma = pltpu.make_async_copy(w_src(w_order[k + 2]),
                                            wbuf.at[slot], w_sems.at[slot])
                dma.start()
                w_dmas.append(dma)
            return slot

        chunks = []
        for d in d_order:
            for j in range(CPB):
                chunks.append((d, j))
        for j in range(CPB):
            chunks.append((0, j))

        def x_row0(d, j):
            tgt = (my + d) % N_DEV
            return tgt * m_per + j * c_rows

        x_dmas = []
        for i in range(2):
            d, j = chunks[i]
            dma = pltpu.make_async_copy(
                x_hbm.at[pl.ds(x_row0(d, j), c_rows), :],
                xf32.at[i % 2], x_sems.at[i % 2])
            dma.start()
            x_dmas.append(dma)

        barrier = pltpu.get_barrier_semaphore()
        for d in range(1, N_DEV):
            pl.semaphore_signal(
                barrier, inc=1,
                device_id=((my + d) % N_DEV,),
                device_id_type=pltpu.DeviceIdType.MESH,
            )
        pl.semaphore_wait(barrier, N_DEV - 1)

        sends = []
        for i, (d, j) in enumerate(chunks):
            x_dmas[i].wait()
            row0 = x_row0(d, j)
            x8_ref[pl.ds(row0, c_rows), :] = (
                xf32[i % 2].astype(jnp.float8_e5m2))
            if i + 2 < len(chunks):
                nd, nj = chunks[i + 2]
                dma = pltpu.make_async_copy(
                    x_hbm.at[pl.ds(x_row0(nd, nj), c_rows), :],
                    xf32.at[i % 2], x_sems.at[i % 2])
                dma.start()
                x_dmas.append(dma)
            if d != 0:
                idx = (d - 1) * CPB + j
                rdma = pltpu.make_async_remote_copy(
                    src_ref=x8_ref.at[pl.ds(row0, c_rows), :],
                    dst_ref=comm_ref.at[d - 1, pl.ds(j * c_rows, c_rows), :],
                    send_sem=send_sems.at[idx],
                    recv_sem=recv_sems.at[idx],
                    device_id=((my + d) % N_DEV,),
                    device_id_type=pltpu.DeviceIdType.MESH,
                )
                rdma.start()
                sends.append(rdma)

        slot = consume_w(0)
        for j in range(CPB):
            rows = pl.ds(j * c_rows, c_rows)
            acc_ref[rows, :] = jnp.dot(
                x8_ref[pl.ds(my * m_per + j * c_rows, c_rows), :],
                w8buf[slot], preferred_element_type=jnp.float32,
            )

        o_dmas = []
        for k, d in enumerate(d_order, start=1):
            slot = consume_w(k)
            for j in range(CPB):
                idx = (d - 1) * CPB + j
                rows = pl.ds(j * c_rows, c_rows)
                recv = pltpu.make_async_remote_copy(
                    src_ref=comm_ref.at[d - 1, rows, :],
                    dst_ref=comm_ref.at[d - 1, rows, :],
                    send_sem=send_sems.at[idx],
                    recv_sem=recv_sems.at[idx],
                    device_id=(my,),
                    device_id_type=pltpu.DeviceIdType.MESH,
                )
                recv.wait_recv()
                a = acc_ref[rows, :] + jnp.dot(
                    comm_ref[d - 1, rows, :], w8buf[slot],
                    preferred_element_type=jnp.float32,
                )
                if k == len(d_order):
                    acc_ref[rows, :] = a * s_ref[0, 0]
                    odma = pltpu.make_async_copy(
                        acc_ref.at[rows, :], out_hbm.at[rows, :],
                        o_sems.at[j])
                    odma.start()
                    o_dmas.append(odma)
                else:
                    acc_ref[rows, :] = a

        for dma in o_dmas:
            dma.wait()
        for rdma in sends:
            rdma.wait_send()

    return pl.pallas_call(
        body,
        out_shape=jax.ShapeDtypeStruct((m_per, N), jnp.float32),
        in_specs=[
            pl.BlockSpec(memory_space=pltpu.MemorySpace.HBM),
            pl.BlockSpec(memory_space=pltpu.MemorySpace.HBM),
            pl.BlockSpec(memory_space=pltpu.SMEM),
        ],
        out_specs=pl.BlockSpec(memory_space=pltpu.MemorySpace.HBM),
        scratch_shapes=[
            pltpu.VMEM((2, c_rows, k_per), jnp.float32),
            pltpu.VMEM((M, k_per), jnp.float8_e5m2),
            pltpu.VMEM((N_DEV - 1, m_per, k_per), jnp.float8_e5m2),
            pltpu.VMEM((2, k_per, N), jnp.float32),
            pltpu.VMEM((2, k_per, N), jnp.float8_e5m2),
            pltpu.VMEM((m_per, N), jnp.float32),
            pltpu.SemaphoreType.DMA(((N_DEV - 1) * CPB,)),
            pltpu.SemaphoreType.DMA(((N_DEV - 1) * CPB,)),
            pltpu.SemaphoreType.DMA((2,)),
            pltpu.SemaphoreType.DMA((2,)),
            pltpu.SemaphoreType.DMA((CPB,)),
        ],
        compiler_params=pltpu.CompilerParams(
            collective_id=0,
            vmem_limit_bytes=100 * 1024 * 1024,
        ),
    )(x, w_mat, s)
